# baseline (speedup 1.0000x reference)
import functools
import math

import jax
import jax.numpy as jnp
from jax.experimental import pallas as pl
from jax.experimental.pallas import tpu as pltpu

D = 256
NH = 128
NL = 128
BQ = 4096
BK = 4096
KBLK = 512
NBLK = BK // KBLK
SCALE = math.sqrt(math.pi / 2.0) / 128.0


def _dot(a, b, dims):
    return jax.lax.dot_general(a.astype(jnp.bfloat16),
                               b.astype(jnp.bfloat16), (dims, ((), ())),
                               preferred_element_type=jnp.float32)


def _nearest(y, c_ref, n):
    if n == 2:
        t = (c_ref[0] + c_ref[1]) * 0.5
        return jnp.where(y > t, c_ref[1], c_ref[0])
    t0 = (c_ref[0] + c_ref[1]) * 0.5
    t1 = (c_ref[1] + c_ref[2]) * 0.5
    t2 = (c_ref[2] + c_ref[3]) * 0.5
    hi = jnp.where(y > t2, c_ref[3], c_ref[2])
    lo = jnp.where(y > t0, c_ref[1], c_ref[0])
    return jnp.where(y > t1, hi, lo)


def _build_chunk(keys, pi_ref, ch_ref, cl_ref, sh_ref, sl_ref):
    vn = jnp.sqrt(jnp.sum(keys * keys, axis=1, keepdims=True))
    kn = keys / (vn + 1e-8)
    parts = []
    for (lo, n_ch, c_ref, n_cent, s_ref) in (
            (0, NH, ch_ref, 4, sh_ref),
            (NH, NL, cl_ref, 2, sl_ref)):
        y = _dot(kn, pi_ref[lo:lo + n_ch, :], (((1,), (1,))))
        y_mse = _nearest(y, c_ref, n_cent)
        resid = y - y_mse
        rnorm = jnp.sqrt(jnp.sum(resid * resid, axis=1, keepdims=True))
        proj = _dot(resid, s_ref[...], (((1,), (1,))))
        signs = jnp.where(proj >= 0.0, 1.0, -1.0)
        corr = _dot(signs, s_ref[...], (((1,), (0,))))
        keff_g = vn * (y_mse + (SCALE * rnorm) * corr)
        parts.append(_dot(keff_g, pi_ref[lo:lo + n_ch, :], (((1,), (0,)))))
    return (parts[0] + parts[1]).astype(jnp.bfloat16)


def _body(ch_ref, cl_ref, q_ref, ka_ref, kb_ref, pi_ref, sh_ref, sl_ref,
          out_ref, k2_ref):
    j = pl.program_id(0)
    half_mm = jax.lax.rem(j, 2)
    half_bd = 1 - half_mm

    @pl.when(j == 0)
    def _init():
        c0 = _build_chunk(kb_ref[...], pi_ref, ch_ref, cl_ref, sh_ref,
                          sl_ref)
        k2_ref[pl.ds(0, KBLK), :] = c0

    prev = k2_ref[pl.ds(pl.multiple_of(half_mm * KBLK, KBLK), KBLK), :]
    @pl.when(j < NBLK - 1)
    def _build_next():
        chunk = _build_chunk(ka_ref[...], pi_ref, ch_ref, cl_ref, sh_ref,
                             sl_ref)
        k2_ref[pl.ds(pl.multiple_of(half_bd * KBLK, KBLK), KBLK), :] = chunk
    out_ref[...] = jax.lax.dot_general(
        q_ref[...].astype(jnp.bfloat16), prev,
        ((((1,), (1,))), ((), ())), preferred_element_type=jnp.float32)


@jax.jit
def kernel(queries, keys, Pi, high_centroids, low_centroids, S_high, S_low):
    est = pl.pallas_call(
        _body,
        grid=(NBLK,),
        in_specs=[
            pl.BlockSpec(memory_space=pltpu.SMEM),
            pl.BlockSpec(memory_space=pltpu.SMEM),
            pl.BlockSpec((BQ, D), lambda j: (0, 0)),
            pl.BlockSpec((KBLK, D),
                         lambda j: (jnp.minimum(j + 1, NBLK - 1), 0)),
            pl.BlockSpec((KBLK, D), lambda j: (0, 0)),
            pl.BlockSpec((D, D), lambda j: (0, 0)),
            pl.BlockSpec((NH, NH), lambda j: (0, 0)),
            pl.BlockSpec((NL, NL), lambda j: (0, 0)),
        ],
        out_specs=pl.BlockSpec((BQ, KBLK), lambda j: (0, j)),
        out_shape=jax.ShapeDtypeStruct((BQ, BK), jnp.float32),
        scratch_shapes=[pltpu.VMEM((2 * KBLK, D), jnp.bfloat16)],
    )(high_centroids, low_centroids, queries, keys, keys, Pi, S_high, S_low)
    return est


# speedup vs baseline: 1.0362x; 1.0362x over previous
import functools
import math

import jax
import jax.numpy as jnp
from jax.experimental import pallas as pl
from jax.experimental.pallas import tpu as pltpu

D = 256
NH = 128
NL = 128
BQ = 4096
BK = 4096
KBLK = 512
NBLK = BK // KBLK
SCALE = math.sqrt(math.pi / 2.0) / 128.0


def _dot(a, b, dims):
    return jax.lax.dot_general(a.astype(jnp.bfloat16),
                               b.astype(jnp.bfloat16), (dims, ((), ())),
                               preferred_element_type=jnp.float32)


def _nearest(y, c_ref, n):
    if n == 2:
        t = (c_ref[0] + c_ref[1]) * 0.5
        return jnp.where(y > t, c_ref[1], c_ref[0])
    t0 = (c_ref[0] + c_ref[1]) * 0.5
    t1 = (c_ref[1] + c_ref[2]) * 0.5
    t2 = (c_ref[2] + c_ref[3]) * 0.5
    hi = jnp.where(y > t2, c_ref[3], c_ref[2])
    lo = jnp.where(y > t0, c_ref[1], c_ref[0])
    return jnp.where(y > t1, hi, lo)


def _build_chunk(keys, pi_ref, ch_ref, cl_ref, sh_ref, sl_ref):
    vn = jnp.sqrt(jnp.sum(keys * keys, axis=1, keepdims=True))
    kn = keys / (vn + 1e-8)
    parts = []
    for (lo, n_ch, c_ref, n_cent, s_ref) in (
            (0, NH, ch_ref, 4, sh_ref),
            (NH, NL, cl_ref, 2, sl_ref)):
        y = _dot(kn, pi_ref[lo:lo + n_ch, :], (((1,), (1,))))
        y_mse = _nearest(y, c_ref, n_cent)
        resid = y - y_mse
        rnorm = jnp.sqrt(jnp.sum(resid * resid, axis=1, keepdims=True))
        proj = _dot(resid, s_ref[...], (((1,), (1,))))
        signs = jnp.where(proj >= 0.0, 1.0, -1.0)
        corr = _dot(signs, s_ref[...], (((1,), (0,))))
        keff_g = vn * (y_mse + (SCALE * rnorm) * corr)
        parts.append(_dot(keff_g, pi_ref[lo:lo + n_ch, :], (((1,), (0,)))))
    return (parts[0] + parts[1]).astype(jnp.bfloat16)


def _body(ch_ref, cl_ref, q_ref, ka_ref, kb_ref, pi_ref, sh_ref, sl_ref,
          out_ref, k2_ref, qb_ref):
    j = pl.program_id(0)
    half_mm = jax.lax.rem(j, 2)
    half_bd = 1 - half_mm

    @pl.when(j == 0)
    def _init():
        qb_ref[...] = q_ref[...].astype(jnp.bfloat16)
        c0 = _build_chunk(kb_ref[...], pi_ref, ch_ref, cl_ref, sh_ref,
                          sl_ref)
        k2_ref[pl.ds(0, KBLK), :] = c0

    prev = k2_ref[pl.ds(pl.multiple_of(half_mm * KBLK, KBLK), KBLK), :]
    chunk = _build_chunk(ka_ref[...], pi_ref, ch_ref, cl_ref, sh_ref, sl_ref)
    k2_ref[pl.ds(pl.multiple_of(half_bd * KBLK, KBLK), KBLK), :] = chunk
    out_ref[...] = jax.lax.dot_general(
        qb_ref[...], prev,
        ((((1,), (1,))), ((), ())), preferred_element_type=jnp.float32)


@jax.jit
def kernel(queries, keys, Pi, high_centroids, low_centroids, S_high, S_low):
    est = pl.pallas_call(
        _body,
        grid=(NBLK,),
        in_specs=[
            pl.BlockSpec(memory_space=pltpu.SMEM),
            pl.BlockSpec(memory_space=pltpu.SMEM),
            pl.BlockSpec((BQ, D), lambda j: (0, 0)),
            pl.BlockSpec((KBLK, D),
                         lambda j: (jnp.minimum(j + 1, NBLK - 1), 0)),
            pl.BlockSpec((KBLK, D), lambda j: (0, 0)),
            pl.BlockSpec((D, D), lambda j: (0, 0)),
            pl.BlockSpec((NH, NH), lambda j: (0, 0)),
            pl.BlockSpec((NL, NL), lambda j: (0, 0)),
        ],
        out_specs=pl.BlockSpec((BQ, KBLK), lambda j: (0, j)),
        out_shape=jax.ShapeDtypeStruct((BQ, BK), jnp.float32),
        scratch_shapes=[pltpu.VMEM((2 * KBLK, D), jnp.bfloat16),
                        pltpu.VMEM((BQ, D), jnp.bfloat16)],
    )(high_centroids, low_centroids, queries, keys, keys, Pi, S_high, S_low)
    return est


# speedup vs baseline: 1.0433x; 1.0069x over previous
import functools
import math

import jax
import jax.numpy as jnp
from jax.experimental import pallas as pl
from jax.experimental.pallas import tpu as pltpu

D = 256
NH = 128
NL = 128
BQ = 4096
BK = 4096
KBLK = 512
NBLK = BK // KBLK
SCALE = math.sqrt(math.pi / 2.0) / 128.0


def _dot(a, b, dims):
    return jax.lax.dot_general(a.astype(jnp.bfloat16),
                               b.astype(jnp.bfloat16), (dims, ((), ())),
                               preferred_element_type=jnp.float32)


def _nearest(y, c_ref, n):
    if n == 2:
        t = (c_ref[0] + c_ref[1]) * 0.5
        return jnp.where(y > t, c_ref[1], c_ref[0])
    t0 = (c_ref[0] + c_ref[1]) * 0.5
    t1 = (c_ref[1] + c_ref[2]) * 0.5
    t2 = (c_ref[2] + c_ref[3]) * 0.5
    hi = jnp.where(y > t2, c_ref[3], c_ref[2])
    lo = jnp.where(y > t0, c_ref[1], c_ref[0])
    return jnp.where(y > t1, hi, lo)


def _build_chunk(keys, pi_ref, ch_ref, cl_ref, sh_ref, sl_ref):
    vn = jnp.sqrt(jnp.sum(keys * keys, axis=1, keepdims=True))
    kn = keys / (vn + 1e-8)
    parts = []
    for (lo, n_ch, c_ref, n_cent, s_ref) in (
            (0, NH, ch_ref, 4, sh_ref),
            (NH, NL, cl_ref, 2, sl_ref)):
        y = _dot(kn, pi_ref[lo:lo + n_ch, :], (((1,), (1,))))
        y_mse = _nearest(y, c_ref, n_cent)
        resid = y - y_mse
        rnorm = jnp.sqrt(jnp.sum(resid * resid, axis=1, keepdims=True))
        proj = _dot(resid, s_ref[...], (((1,), (1,))))
        signs = jnp.where(proj >= 0.0, 1.0, -1.0)
        corr = _dot(signs, s_ref[...], (((1,), (0,))))
        keff_g = vn * (y_mse + (SCALE * rnorm) * corr)
        parts.append(_dot(keff_g, pi_ref[lo:lo + n_ch, :], (((1,), (0,)))))
    return (parts[0] + parts[1]).astype(jnp.bfloat16)


def _body(ch_ref, cl_ref, q_ref, ka_ref, kb_ref, pi_ref, sh_ref, sl_ref,
          out_ref, k2_ref, qb_ref, qf_ref, sem):
    j = pl.program_id(0)
    half_mm = jax.lax.rem(j, 2)
    half_bd = 1 - half_mm

    @pl.when(j == 0)
    def _init():
        cp = pltpu.make_async_copy(q_ref, qf_ref, sem)
        cp.start()
        c0 = _build_chunk(kb_ref[...], pi_ref, ch_ref, cl_ref, sh_ref,
                          sl_ref)
        k2_ref[pl.ds(0, KBLK), :] = c0
        cp.wait()
        qb_ref[...] = qf_ref[...].astype(jnp.bfloat16)

    prev = k2_ref[pl.ds(pl.multiple_of(half_mm * KBLK, KBLK), KBLK), :]
    chunk = _build_chunk(ka_ref[...], pi_ref, ch_ref, cl_ref, sh_ref, sl_ref)
    k2_ref[pl.ds(pl.multiple_of(half_bd * KBLK, KBLK), KBLK), :] = chunk
    out_ref[...] = jax.lax.dot_general(
        qb_ref[...], prev,
        ((((1,), (1,))), ((), ())), preferred_element_type=jnp.float32)


@jax.jit
def kernel(queries, keys, Pi, high_centroids, low_centroids, S_high, S_low):
    est = pl.pallas_call(
        _body,
        grid=(NBLK,),
        in_specs=[
            pl.BlockSpec(memory_space=pltpu.SMEM),
            pl.BlockSpec(memory_space=pltpu.SMEM),
            pl.BlockSpec(memory_space=pl.ANY),
            pl.BlockSpec((KBLK, D),
                         lambda j: (jnp.minimum(j + 1, NBLK - 1), 0)),
            pl.BlockSpec((KBLK, D), lambda j: (0, 0)),
            pl.BlockSpec((D, D), lambda j: (0, 0)),
            pl.BlockSpec((NH, NH), lambda j: (0, 0)),
            pl.BlockSpec((NL, NL), lambda j: (0, 0)),
        ],
        out_specs=pl.BlockSpec((BQ, KBLK), lambda j: (0, j)),
        out_shape=jax.ShapeDtypeStruct((BQ, BK), jnp.float32),
        scratch_shapes=[pltpu.VMEM((2 * KBLK, D), jnp.bfloat16),
                        pltpu.VMEM((BQ, D), jnp.bfloat16),
                        pltpu.VMEM((BQ, D), jnp.float32),
                        pltpu.SemaphoreType.DMA],
    )(high_centroids, low_centroids, queries, keys, keys, Pi, S_high, S_low)
    return est
